# baseline (speedup 1.0000x reference)
import functools

import jax
import jax.numpy as jnp
from jax import lax
from jax.experimental import pallas as pl
from jax.experimental.pallas import tpu as pltpu
from jax.experimental.pallas import tpu_sc as plsc


def _sc_permute(z, perm, *, B, C, H, W, NC, NS):
    CHALF = C // 2

    mesh = plsc.VectorSubcoreMesh(core_axis_name="c", subcore_axis_name="s")

    @functools.partial(
        pl.kernel,
        mesh=mesh,
        out_type=jax.ShapeDtypeStruct((B, C, H, W), jnp.float32),
        compiler_params=pltpu.CompilerParams(
            use_tc_tiling_on_sc=True, needs_layout_passes=False
        ),
        scratch_types=[
            pltpu.VMEM((C,), jnp.int32),
            pltpu.VMEM((2, 1, 1, H, W), jnp.float32),
            pltpu.SemaphoreType.DMA,
            pltpu.SemaphoreType.DMA,
            pltpu.SemaphoreType.DMA,
            pltpu.SemaphoreType.DMA,
        ],
    )
    def sc_copy(z_hbm, perm_hbm, out_hbm, perm_v, buf_v, g0, g1, s0, s1):
        gsem = (g0, g1)
        ssem = (s0, s1)
        wid = lax.axis_index("s") * NC + lax.axis_index("c")
        pltpu.sync_copy(perm_hbm, perm_v)
        b = wid // 2
        c0 = (wid % 2) * CHALF

        def perm_at(c):
            vec = perm_v[pl.ds((c // 16) * 16, 16)]
            lane = lax.iota(jnp.int32, 16)
            return jnp.sum(jnp.where(lane == c % 16, vec, 0))

        def scatter_start(c, bb):
            pltpu.make_async_copy(
                buf_v.at[bb], out_hbm.at[pl.ds(b, 1), pl.ds(c, 1)], ssem[bb]
            ).start()

        def scatter_wait(bb):
            pltpu.make_async_copy(
                buf_v.at[bb], out_hbm.at[pl.ds(0, 1), pl.ds(0, 1)], ssem[bb]
            ).wait()

        def body(p, _):
            for bb in range(2):
                i = 2 * p + bb
                c = c0 + i

                @pl.when(p > 0)
                def _():
                    scatter_wait(bb)

                pc = perm_at(c)
                pltpu.async_copy(
                    z_hbm.at[pl.ds(b, 1), pl.ds(pc, 1)], buf_v.at[bb], gsem[bb]
                ).wait()
                scatter_start(c, bb)
            return 0

        lax.fori_loop(0, CHALF // 2, body, 0)
        scatter_wait(0)
        scatter_wait(1)

    return sc_copy(z, perm)


def kernel(z, perm):
    B, C, H, W = z.shape
    info = plsc.get_sparse_core_info()
    out = _sc_permute(
        z,
        perm.astype(jnp.int32),
        B=B,
        C=C,
        H=H,
        W=W,
        NC=info.num_cores,
        NS=info.num_subcores,
    )
    return out, jnp.zeros((), z.dtype)
